# baseline (speedup 1.0000x reference)
import numpy as np

import jax
import jax.numpy as jnp
from jax.experimental import pallas as pl
from jax.experimental.pallas import tpu as pltpu

_P = 16

_ROT_A = (13, 15, 26, 6)
_ROT_B = (17, 29, 16, 24)

_LO = np.nextafter(np.float32(-1.0), np.float32(0.0), dtype=np.float32)
_SPAN = np.float32(np.float32(1.0) - _LO)
_SQRT2 = np.float32(np.sqrt(2.0))

_ERFINV_SMALL = (2.81022636e-08, 3.43273939e-07, -3.5233877e-06,
                 -4.39150654e-06, 0.00021858087, -0.00125372503,
                 -0.00417768164, 0.246640727, 1.50140941)
_ERFINV_BIG = (-0.000200214257, 0.000100950558, 0.00134934322,
               -0.00367342844, 0.00573950773, -0.0076224613,
               0.00943887047, 1.00167406, 2.83297682)


def _rotl(x, r):
    return jax.lax.shift_left(x, jnp.uint32(r)) | jax.lax.shift_right_logical(
        x, jnp.uint32(32 - r))


def _threefry_bits(k0, k1, ks2, x1):
    x0 = k0
    inject = ((k1, ks2), (ks2, k0), (k0, k1), (k1, ks2), (ks2, k0))
    for i, rots in enumerate((_ROT_A, _ROT_B, _ROT_A, _ROT_B, _ROT_A)):
        for r in rots:
            x0 = x0 + x1
            x1 = _rotl(x1, r)
            x1 = x0 ^ x1
        a, c = inject[i]
        x0 = x0 + a
        x1 = x1 + (c + jnp.uint32(i + 1))
    return x0 ^ x1


def _eps_from_bits(bits):
    fb = jax.lax.shift_right_logical(bits, jnp.uint32(9)) | jnp.uint32(
        0x3F800000)
    u01 = pltpu.bitcast(fb, jnp.float32) - jnp.float32(1.0)
    u = jnp.maximum(jnp.float32(_LO), u01 * jnp.float32(_SPAN)
                    + jnp.float32(_LO))
    w = -jnp.log1p(-u * u)
    ws = w - jnp.float32(2.5)
    wb = jnp.sqrt(w) - jnp.float32(3.0)
    ps = jnp.float32(_ERFINV_SMALL[0])
    pb = jnp.float32(_ERFINV_BIG[0])
    for cs, cvb in zip(_ERFINV_SMALL[1:], _ERFINV_BIG[1:]):
        ps = jnp.float32(cs) + ps * ws
        pb = jnp.float32(cvb) + pb * wb
    p = jnp.where(w < jnp.float32(5.0), ps, pb)
    return jnp.float32(_SQRT2) * (p * u)


def _eps_from_counts(k0, k1, ks2, lin_plus_k1):
    return _eps_from_bits(
        _threefry_bits(k0, k1, ks2, lin_plus_k1.astype(jnp.uint32)))


def _vad_kernel(idx_ref, kd_ref, tab_hbm, idxv_ref,
                mu_ref, lv_ref, std_ref, lat_ref, aug_ref,
                chunks, sem):
    tb = mu_ref.shape[0]
    dim = mu_ref.shape[1]
    nrows = tab_hbm.shape[0]
    row_elems = (_P + 1) * dim
    base_row = pl.program_id(0) * tb
    k0 = kd_ref[0].astype(jnp.uint32)
    k1 = kd_ref[1].astype(jnp.uint32)
    k1i = kd_ref[1]
    ks2 = k0 ^ k1 ^ jnp.uint32(0x1BD11BDA)

    for i in range(tb):
        r = jnp.clip(idx_ref[base_row + i], 0, nrows - 1)
        c = pl.multiple_of((r >> 3) << 3, 8)
        pltpu.make_async_copy(tab_hbm.at[pl.ds(c, 8), :],
                              chunks.at[pl.ds(i * 8, 8), :], sem).start()

    hb = max(tb // 4, 1)
    for h in range(tb // hb):
        lin = (jax.lax.broadcasted_iota(jnp.int32, (hb, dim), 0) * row_elems
               + jax.lax.broadcasted_iota(jnp.int32, (hb, dim), 1)
               + ((base_row + h * hb) * row_elems + _P * dim + k1i))
        lat_ref[pl.ds(h * hb, hb), :] = _eps_from_counts(k0, k1, ks2, lin)

    pltpu.make_async_copy(tab_hbm.at[pl.ds(0, 8 * tb), :],
                          chunks.at[pl.ds(0, 8 * tb), :], sem).wait()
    eb = min(tb, 128)
    parts = []
    for e0 in range(0, tb, eb):
        pos = ((idxv_ref[pl.ds(e0, eb), :] & 7)
               + 8 * jax.lax.broadcasted_iota(jnp.int32, (eb, 1), 0))
        lane = jax.lax.broadcasted_iota(jnp.int32, (eb, 8 * eb), 1)
        onehot = (lane == pos).astype(jnp.float32)
        parts.append(jax.lax.dot_general(
            onehot, chunks[pl.ds(e0 * 8, eb * 8), :],
            (((1,), (0,)), ((), ())),
            precision=jax.lax.Precision.HIGHEST,
            preferred_element_type=jnp.float32))
    picked = jnp.concatenate(parts, axis=0) if len(parts) > 1 else parts[0]
    mu = picked[:, :dim]
    logvar = picked[:, dim:]
    std = jnp.exp(0.5 * logvar)
    mu_ref[...] = mu
    lv_ref[...] = logvar
    std_ref[...] = std
    lat_ref[...] = mu + lat_ref[...] * std

    half = _P // 2

    def chunk_bits(j):
        b0 = (j >> 1) * 8
        p0 = (j & 1) * half
        shp = (8, half, dim)
        lin = (jax.lax.broadcasted_iota(jnp.int32, shp, 0) * row_elems
               + jax.lax.broadcasted_iota(jnp.int32, shp, 1) * dim
               + jax.lax.broadcasted_iota(jnp.int32, shp, 2)
               + ((base_row + b0) * row_elems + p0 * dim + k1i))
        return _threefry_bits(k0, k1, ks2, lin.astype(jnp.uint32))

    def aug_body(j, carry):
        b0 = pl.multiple_of((j >> 1) * 8, 8)
        p0 = pl.multiple_of((j & 1) * half, half)
        nxt = chunk_bits(j + 1)
        eps = _eps_from_bits(carry)
        mu8 = mu_ref[pl.ds(b0, 8), :]
        std8 = std_ref[pl.ds(b0, 8), :]
        aug_ref[pl.ds(b0, 8), pl.ds(p0, half), :] = (
            mu8[:, None, :] + eps * std8[:, None, :])
        return nxt

    jax.lax.fori_loop(0, (tb // 8) * 2, aug_body, chunk_bits(0))


def kernel(idx, tab_fused, eps_seed):
    b = int(idx.shape[0])
    n_pad, two_dim = tab_fused.shape
    dim = two_dim // 2

    kd = jax.lax.bitcast_convert_type(
        jax.random.key_data(jax.random.key(eps_seed)), jnp.int32)

    idx32 = idx.astype(jnp.int32)
    idx_col = idx32.reshape(b, 1)

    nsteps = 1
    tb = b

    grid_spec = pltpu.PrefetchScalarGridSpec(
        num_scalar_prefetch=2,
        grid=(nsteps,),
        in_specs=[
            pl.BlockSpec(memory_space=pl.ANY),
            pl.BlockSpec((tb, 1), lambda g, *_: (g, 0)),
        ],
        out_specs=[
            pl.BlockSpec((tb, dim), lambda g, *_: (g, 0)),
            pl.BlockSpec((tb, dim), lambda g, *_: (g, 0)),
            pl.BlockSpec((tb, dim), lambda g, *_: (g, 0)),
            pl.BlockSpec((tb, dim), lambda g, *_: (g, 0)),
            pl.BlockSpec((tb, _P, dim), lambda g, *_: (g, 0, 0)),
        ],
        scratch_shapes=[
            pltpu.VMEM((tb * 8, two_dim), jnp.float32),
            pltpu.SemaphoreType.DMA,
        ],
    )
    out_shape = (tuple(jax.ShapeDtypeStruct((b, dim), jnp.float32)
                       for _ in range(4))
                 + (jax.ShapeDtypeStruct((b, _P, dim), jnp.float32),))
    mu, logvar, std, latent, latent_aug = pl.pallas_call(
        _vad_kernel,
        grid_spec=grid_spec,
        out_shape=out_shape,
        compiler_params=pltpu.CompilerParams(
            dimension_semantics=("arbitrary",)),
    )(idx32, kd, tab_fused, idx_col)

    return {'latent_code': latent,
            'latent_code_augment': latent_aug,
            'mu': mu, 'logvar': logvar, 'std': std}


# speedup vs baseline: 1.0358x; 1.0358x over previous
import numpy as np

import jax
import jax.numpy as jnp
from jax.experimental import pallas as pl
from jax.experimental.pallas import tpu as pltpu

_P = 16

_ROT_A = (13, 15, 26, 6)
_ROT_B = (17, 29, 16, 24)

_LO = np.nextafter(np.float32(-1.0), np.float32(0.0), dtype=np.float32)
_SPAN = np.float32(np.float32(1.0) - _LO)
_SQRT2 = np.float32(np.sqrt(2.0))

_ERFINV_SMALL = (2.81022636e-08, 3.43273939e-07, -3.5233877e-06,
                 -4.39150654e-06, 0.00021858087, -0.00125372503,
                 -0.00417768164, 0.246640727, 1.50140941)
_ERFINV_BIG = (-0.000200214257, 0.000100950558, 0.00134934322,
               -0.00367342844, 0.00573950773, -0.0076224613,
               0.00943887047, 1.00167406, 2.83297682)


def _rotl(x, r):
    return jax.lax.shift_left(x, jnp.uint32(r)) | jax.lax.shift_right_logical(
        x, jnp.uint32(32 - r))


def _threefry_bits(k0, k1, ks2, x1):
    x0 = k0
    inject = ((k1, ks2), (ks2, k0), (k0, k1), (k1, ks2), (ks2, k0))
    for i, rots in enumerate((_ROT_A, _ROT_B, _ROT_A, _ROT_B, _ROT_A)):
        for r in rots:
            x0 = x0 + x1
            x1 = _rotl(x1, r)
            x1 = x0 ^ x1
        a, c = inject[i]
        x0 = x0 + a
        x1 = x1 + (c + jnp.uint32(i + 1))
    return x0 ^ x1


def _eps_from_bits(bits):
    fb = jax.lax.shift_right_logical(bits, jnp.uint32(9)) | jnp.uint32(
        0x3F800000)
    u01 = pltpu.bitcast(fb, jnp.float32) - jnp.float32(1.0)
    u = jnp.maximum(jnp.float32(_LO), u01 * jnp.float32(_SPAN)
                    + jnp.float32(_LO))
    w = -jnp.log1p(-u * u)
    ws = w - jnp.float32(2.5)
    wb = jnp.sqrt(w) - jnp.float32(3.0)
    ps = jnp.float32(_ERFINV_SMALL[0])
    pb = jnp.float32(_ERFINV_BIG[0])
    for cs, cvb in zip(_ERFINV_SMALL[1:], _ERFINV_BIG[1:]):
        ps = jnp.float32(cs) + ps * ws
        pb = jnp.float32(cvb) + pb * wb
    p = jnp.where(w < jnp.float32(5.0), ps, pb)
    return jnp.float32(_SQRT2) * (p * u)


def _eps_from_counts(k0, k1, ks2, lin_plus_k1):
    return _eps_from_bits(
        _threefry_bits(k0, k1, ks2, lin_plus_k1.astype(jnp.uint32)))


def _vad_kernel(idx_ref, kd_ref, tab_hbm, idxv_ref,
                mu_ref, lv_ref, std_ref, lat_ref, aug_ref,
                chunks, sem):
    tb = mu_ref.shape[0]
    dim = mu_ref.shape[1]
    nrows = tab_hbm.shape[0]
    row_elems = (_P + 1) * dim
    base_row = pl.program_id(0) * tb
    k0 = kd_ref[0].astype(jnp.uint32)
    k1 = kd_ref[1].astype(jnp.uint32)
    k1i = kd_ref[1]
    ks2 = k0 ^ k1 ^ jnp.uint32(0x1BD11BDA)

    for i in range(tb):
        r = jnp.clip(idx_ref[base_row + i], 0, nrows - 1)
        c = pl.multiple_of((r >> 3) << 3, 8)
        pltpu.make_async_copy(tab_hbm.at[pl.ds(c, 8), :],
                              chunks.at[pl.ds(i * 8, 8), :], sem).start()

    hb = max(tb // 4, 1)
    for h in range(tb // hb):
        lin = (jax.lax.broadcasted_iota(jnp.int32, (hb, dim), 0) * row_elems
               + jax.lax.broadcasted_iota(jnp.int32, (hb, dim), 1)
               + ((base_row + h * hb) * row_elems + _P * dim + k1i))
        lat_ref[pl.ds(h * hb, hb), :] = _eps_from_counts(k0, k1, ks2, lin)

    pltpu.make_async_copy(tab_hbm.at[pl.ds(0, 8 * tb), :],
                          chunks.at[pl.ds(0, 8 * tb), :], sem).wait()
    eb = min(tb, 128)
    parts = []
    for e0 in range(0, tb, eb):
        pos = ((idxv_ref[pl.ds(e0, eb), :] & 7)
               + 8 * jax.lax.broadcasted_iota(jnp.int32, (eb, 1), 0))
        lane = jax.lax.broadcasted_iota(jnp.int32, (eb, 8 * eb), 1)
        onehot = (lane == pos).astype(jnp.float32)
        parts.append(jax.lax.dot_general(
            onehot, chunks[pl.ds(e0 * 8, eb * 8), :],
            (((1,), (0,)), ((), ())),
            precision=jax.lax.Precision.HIGHEST,
            preferred_element_type=jnp.float32))
    picked = jnp.concatenate(parts, axis=0) if len(parts) > 1 else parts[0]
    mu = picked[:, :dim]
    logvar = picked[:, dim:]
    std = jnp.exp(0.5 * logvar)
    mu_ref[...] = mu
    lv_ref[...] = logvar
    std_ref[...] = std
    lat_ref[...] = mu + lat_ref[...] * std

    def block_bits(bi):
        outs = []
        for p0 in (0, _P // 2):
            shp = (8, _P // 2, dim)
            lin = (jax.lax.broadcasted_iota(jnp.int32, shp, 0) * row_elems
                   + jax.lax.broadcasted_iota(jnp.int32, shp, 1) * dim
                   + jax.lax.broadcasted_iota(jnp.int32, shp, 2)
                   + ((base_row + bi * 8) * row_elems + p0 * dim + k1i))
            outs.append(_threefry_bits(k0, k1, ks2,
                                       lin.astype(jnp.uint32)))
        return tuple(outs)

    def aug_body(i, carry):
        b0 = pl.multiple_of(i * 8, 8)
        mu8 = mu_ref[pl.ds(b0, 8), :]
        std8 = std_ref[pl.ds(b0, 8), :]
        nxt = block_bits(i + 1)
        for p0, bits in ((0, carry[0]), (_P // 2, carry[1])):
            eps = _eps_from_bits(bits)
            aug_ref[pl.ds(b0, 8), pl.ds(p0, _P // 2), :] = (
                mu8[:, None, :] + eps * std8[:, None, :])
        return nxt

    jax.lax.fori_loop(0, tb // 8, aug_body, block_bits(0))


def kernel(idx, tab_fused, eps_seed):
    b = int(idx.shape[0])
    n_pad, two_dim = tab_fused.shape
    dim = two_dim // 2

    kd = jax.lax.bitcast_convert_type(
        jax.random.key_data(jax.random.key(eps_seed)), jnp.int32)

    idx32 = idx.astype(jnp.int32)
    idx_col = idx32.reshape(b, 1)

    nsteps = 1
    tb = b

    grid_spec = pltpu.PrefetchScalarGridSpec(
        num_scalar_prefetch=2,
        grid=(nsteps,),
        in_specs=[
            pl.BlockSpec(memory_space=pl.ANY),
            pl.BlockSpec((tb, 1), lambda g, *_: (g, 0)),
        ],
        out_specs=[
            pl.BlockSpec((tb, dim), lambda g, *_: (g, 0)),
            pl.BlockSpec((tb, dim), lambda g, *_: (g, 0)),
            pl.BlockSpec((tb, dim), lambda g, *_: (g, 0)),
            pl.BlockSpec((tb, dim), lambda g, *_: (g, 0)),
            pl.BlockSpec((tb, _P, dim), lambda g, *_: (g, 0, 0)),
        ],
        scratch_shapes=[
            pltpu.VMEM((tb * 8, two_dim), jnp.float32),
            pltpu.SemaphoreType.DMA,
        ],
    )
    out_shape = (tuple(jax.ShapeDtypeStruct((b, dim), jnp.float32)
                       for _ in range(4))
                 + (jax.ShapeDtypeStruct((b, _P, dim), jnp.float32),))
    mu, logvar, std, latent, latent_aug = pl.pallas_call(
        _vad_kernel,
        grid_spec=grid_spec,
        out_shape=out_shape,
        compiler_params=pltpu.CompilerParams(
            dimension_semantics=("arbitrary",)),
    )(idx32, kd, tab_fused, idx_col)

    return {'latent_code': latent,
            'latent_code_augment': latent_aug,
            'mu': mu, 'logvar': logvar, 'std': std}


# speedup vs baseline: 1.0406x; 1.0046x over previous
import numpy as np

import jax
import jax.numpy as jnp
from jax.experimental import pallas as pl
from jax.experimental.pallas import tpu as pltpu

_P = 16

_ROT_A = (13, 15, 26, 6)
_ROT_B = (17, 29, 16, 24)

_LO = np.nextafter(np.float32(-1.0), np.float32(0.0), dtype=np.float32)
_SPAN = np.float32(np.float32(1.0) - _LO)
_SQRT2 = np.float32(np.sqrt(2.0))

_ERFINV_SMALL = (2.81022636e-08, 3.43273939e-07, -3.5233877e-06,
                 -4.39150654e-06, 0.00021858087, -0.00125372503,
                 -0.00417768164, 0.246640727, 1.50140941)
_ERFINV_BIG = (-0.000200214257, 0.000100950558, 0.00134934322,
               -0.00367342844, 0.00573950773, -0.0076224613,
               0.00943887047, 1.00167406, 2.83297682)


def _rotl(x, r):
    return jax.lax.shift_left(x, jnp.uint32(r)) | jax.lax.shift_right_logical(
        x, jnp.uint32(32 - r))


def _threefry_bits(k0, k1, ks2, x1):
    x0 = k0
    inject = ((k1, ks2), (ks2, k0), (k0, k1), (k1, ks2), (ks2, k0))
    for i, rots in enumerate((_ROT_A, _ROT_B, _ROT_A, _ROT_B, _ROT_A)):
        for r in rots:
            x0 = x0 + x1
            x1 = _rotl(x1, r)
            x1 = x0 ^ x1
        a, c = inject[i]
        x0 = x0 + a
        x1 = x1 + (c + jnp.uint32(i + 1))
    return x0 ^ x1


def _eps_from_bits(bits):
    fb = jax.lax.shift_right_logical(bits, jnp.uint32(9)) | jnp.uint32(
        0x3F800000)
    u01 = pltpu.bitcast(fb, jnp.float32) - jnp.float32(1.0)
    u = jnp.maximum(jnp.float32(_LO), u01 * jnp.float32(_SPAN)
                    + jnp.float32(_LO))
    w = -jnp.log1p(-u * u)
    ws = w - jnp.float32(2.5)
    wb = jnp.sqrt(w) - jnp.float32(3.0)
    ps = jnp.float32(_ERFINV_SMALL[0])
    pb = jnp.float32(_ERFINV_BIG[0])
    for cs, cvb in zip(_ERFINV_SMALL[1:], _ERFINV_BIG[1:]):
        ps = jnp.float32(cs) + ps * ws
        pb = jnp.float32(cvb) + pb * wb
    p = jnp.where(w < jnp.float32(5.0), ps, pb)
    return jnp.float32(_SQRT2) * (p * u)


def _eps_from_counts(k0, k1, ks2, lin_plus_k1):
    return _eps_from_bits(
        _threefry_bits(k0, k1, ks2, lin_plus_k1.astype(jnp.uint32)))


def _vad_kernel(idx_ref, kd_ref, tab_hbm, idxv_ref,
                mu_ref, lv_ref, std_ref, lat_ref, aug_ref,
                chunks, sem):
    tb = mu_ref.shape[0]
    dim = mu_ref.shape[1]
    nrows = tab_hbm.shape[0]
    row_elems = (_P + 1) * dim
    base_row = pl.program_id(0) * tb
    k0 = kd_ref[0].astype(jnp.uint32)
    k1 = kd_ref[1].astype(jnp.uint32)
    k1i = kd_ref[1]
    ks2 = k0 ^ k1 ^ jnp.uint32(0x1BD11BDA)

    pow2 = (nrows & (nrows - 1)) == 0
    for i in range(tb):
        r = idx_ref[base_row + i]
        r = (r & (nrows - 1)) if pow2 else jnp.clip(r, 0, nrows - 1)
        c = pl.multiple_of((r >> 3) << 3, 8)
        pltpu.make_async_copy(tab_hbm.at[pl.ds(c, 8), :],
                              chunks.at[pl.ds(i * 8, 8), :], sem).start()

    hb = max(tb // 4, 1)
    for h in range(tb // hb):
        lin = (jax.lax.broadcasted_iota(jnp.int32, (hb, dim), 0) * row_elems
               + jax.lax.broadcasted_iota(jnp.int32, (hb, dim), 1)
               + ((base_row + h * hb) * row_elems + _P * dim + k1i))
        lat_ref[pl.ds(h * hb, hb), :] = _eps_from_counts(k0, k1, ks2, lin)

    pltpu.make_async_copy(tab_hbm.at[pl.ds(0, 8 * tb), :],
                          chunks.at[pl.ds(0, 8 * tb), :], sem).wait()
    eb = min(tb, 128)
    parts = []
    for e0 in range(0, tb, eb):
        pos = ((idxv_ref[pl.ds(e0, eb), pl.ds(0, 1)] & 7)
               + 8 * jax.lax.broadcasted_iota(jnp.int32, (eb, 1), 0))
        lane = jax.lax.broadcasted_iota(jnp.int32, (eb, 8 * eb), 1)
        onehot = (lane == pos).astype(jnp.float32)
        parts.append(jax.lax.dot_general(
            onehot, chunks[pl.ds(e0 * 8, eb * 8), :],
            (((1,), (0,)), ((), ())),
            precision=jax.lax.Precision.HIGHEST,
            preferred_element_type=jnp.float32))
    picked = jnp.concatenate(parts, axis=0) if len(parts) > 1 else parts[0]
    mu = picked[:, :dim]
    logvar = picked[:, dim:]
    std = jnp.exp(0.5 * logvar)
    mu_ref[...] = mu
    lv_ref[...] = logvar
    std_ref[...] = std
    lat_ref[...] = mu + lat_ref[...] * std

    def block_bits(bi):
        outs = []
        for p0 in (0, _P // 2):
            shp = (8, _P // 2, dim)
            lin = (jax.lax.broadcasted_iota(jnp.int32, shp, 0) * row_elems
                   + jax.lax.broadcasted_iota(jnp.int32, shp, 1) * dim
                   + jax.lax.broadcasted_iota(jnp.int32, shp, 2)
                   + ((base_row + bi * 8) * row_elems + p0 * dim + k1i))
            outs.append(_threefry_bits(k0, k1, ks2,
                                       lin.astype(jnp.uint32)))
        return tuple(outs)

    def aug_body(i, carry):
        b0 = pl.multiple_of(i * 8, 8)
        mu8 = mu_ref[pl.ds(b0, 8), :]
        std8 = std_ref[pl.ds(b0, 8), :]
        nxt = block_bits(i + 1)
        for p0, bits in ((0, carry[0]), (_P // 2, carry[1])):
            eps = _eps_from_bits(bits)
            aug_ref[pl.ds(b0, 8), pl.ds(p0, _P // 2), :] = (
                mu8[:, None, :] + eps * std8[:, None, :])
        return nxt

    jax.lax.fori_loop(0, tb // 8, aug_body, block_bits(0))


def kernel(idx, tab_fused, eps_seed):
    b = int(idx.shape[0])
    n_pad, two_dim = tab_fused.shape
    dim = two_dim // 2

    kd = jax.lax.bitcast_convert_type(
        jax.random.key_data(jax.random.key(eps_seed)), jnp.int32)

    idx32 = idx.astype(jnp.int32)
    idx_col = jnp.broadcast_to(idx32[:, None], (b, 128))

    nsteps = 1
    tb = b

    grid_spec = pltpu.PrefetchScalarGridSpec(
        num_scalar_prefetch=2,
        grid=(nsteps,),
        in_specs=[
            pl.BlockSpec(memory_space=pl.ANY),
            pl.BlockSpec((tb, 128), lambda g, *_: (g, 0)),
        ],
        out_specs=[
            pl.BlockSpec((tb, dim), lambda g, *_: (g, 0)),
            pl.BlockSpec((tb, dim), lambda g, *_: (g, 0)),
            pl.BlockSpec((tb, dim), lambda g, *_: (g, 0)),
            pl.BlockSpec((tb, dim), lambda g, *_: (g, 0)),
            pl.BlockSpec((tb, _P, dim), lambda g, *_: (g, 0, 0)),
        ],
        scratch_shapes=[
            pltpu.VMEM((tb * 8, two_dim), jnp.float32),
            pltpu.SemaphoreType.DMA,
        ],
    )
    out_shape = (tuple(jax.ShapeDtypeStruct((b, dim), jnp.float32)
                       for _ in range(4))
                 + (jax.ShapeDtypeStruct((b, _P, dim), jnp.float32),))
    mu, logvar, std, latent, latent_aug = pl.pallas_call(
        _vad_kernel,
        grid_spec=grid_spec,
        out_shape=out_shape,
        compiler_params=pltpu.CompilerParams(
            dimension_semantics=("arbitrary",)),
    )(idx32, kd, tab_fused, idx_col)

    return {'latent_code': latent,
            'latent_code_augment': latent_aug,
            'mu': mu, 'logvar': logvar, 'std': std}
